# baseline (speedup 1.0000x reference)
import math

import jax
import jax.numpy as jnp
from jax.experimental import pallas as pl
from jax.experimental.pallas import tpu as pltpu

B = 64
T = 1024
D = 256
BT = 8
NS = 2
TS = T // 2 // NS


def _ln(x, g, b, eps=1e-5):
    mu = jnp.mean(x, axis=-1, keepdims=True)
    var = jnp.mean((x - mu) * (x - mu), axis=-1, keepdims=True)
    return (x - mu) * jax.lax.rsqrt(var + eps) * g + b


def _body(k1, k2, v1, v2, q_ref, Wq_ref, Wk_ref, Wv_ref, Wo_ref,
          bq_ref, bv_ref, bo_ref, gq_ref, bqn_ref, go_ref, bon_ref,
          out_ref, m_s, l_s, rv_s):
    tc = pl.program_id(1)

    qn = _ln(q_ref[:], gq_ref[:], bqn_ref[:])
    qp = jax.lax.dot_general(qn, Wq_ref[:], (((1,), (1,)), ((), ())),
                             preferred_element_type=jnp.float32) + bq_ref[:]
    qk = jax.lax.dot_general(qp, Wk_ref[:], (((1,), (0,)), ((), ())),
                             preferred_element_type=jnp.float32)
    qk = qk * (1.0 / math.sqrt(D))

    dn = (((1,), (2,)), ((0,), (0,)))
    ss = [jax.lax.dot_general(qk, kr[:], dn,
                              preferred_element_type=jnp.float32)
          for kr in (k1, k2)]
    m_c = jnp.maximum(ss[0].max(axis=1, keepdims=True),
                      ss[1].max(axis=1, keepdims=True))
    es = [jnp.exp(si - m_c) for si in ss]
    s_c = es[0].sum(axis=1, keepdims=True) + es[1].sum(axis=1, keepdims=True)
    dn2 = (((1,), (1,)), ((0,), (0,)))
    rv_c = (jax.lax.dot_general(es[0], v1[:], dn2,
                                preferred_element_type=jnp.float32)
            + jax.lax.dot_general(es[1], v2[:], dn2,
                                  preferred_element_type=jnp.float32))

    @pl.when(tc == 0)
    def _():
        m_s[:, 0:1] = m_c
        l_s[:, 0:1] = s_c
        rv_s[:] = rv_c

    @pl.when(tc == 1)
    def _():
        m0 = m_s[:, 0:1]
        l0 = l_s[:, 0:1]
        mn = jnp.maximum(m0, m_c)
        a = jnp.exp(m0 - mn)
        c = jnp.exp(m_c - mn)
        rv = rv_s[:] * a + rv_c * c
        s = l0 * a + s_c * c
        rv = rv * (1.0 / s)
        ret = jax.lax.dot_general(rv, Wv_ref[:], (((1,), (1,)), ((), ())),
                                  preferred_element_type=jnp.float32) \
            + bv_ref[:]
        out = jax.lax.dot_general(ret, Wo_ref[:], (((1,), (1,)), ((), ())),
                                  preferred_element_type=jnp.float32) \
            + bo_ref[:]
        out_ref[:] = _ln(out, go_ref[:], bon_ref[:])


def kernel(keys_in, values_in, query, Wq, bq, Wk, bk, Wv, bv, Wo, bo,
           g_q, b_qn, g_o, b_on):
    del bk
    vecs = [v.reshape(1, D) for v in (bq, bv, bo, g_q, b_qn, g_o, b_on)]
    full = pl.BlockSpec((1, D), lambda i, tc: (0, 0))
    mat = pl.BlockSpec((D, D), lambda i, tc: (0, 0))

    def win(j):
        return pl.BlockSpec((BT, TS, D),
                            lambda i, tc, j=j: (i, tc * NS + j, 0))

    kv_specs = [win(j) for j in range(NS)] * 2
    out = pl.pallas_call(
        _body,
        grid=(B // BT, 2),
        in_specs=kv_specs + [
            pl.BlockSpec((BT, D), lambda i, tc: (i, 0)),
            mat, mat, mat, mat,
            full, full, full, full, full, full, full,
        ],
        out_specs=pl.BlockSpec((BT, D), lambda i, tc: (i, 0)),
        out_shape=jax.ShapeDtypeStruct((B, D), jnp.float32),
        scratch_shapes=[
            pltpu.VMEM((BT, 128), jnp.float32),
            pltpu.VMEM((BT, 128), jnp.float32),
            pltpu.VMEM((BT, D), jnp.float32),
        ],
    )(*([keys_in] * NS), *([values_in] * NS), query,
      Wq, Wk, Wv, Wo, *vecs)
    return out


# speedup vs baseline: 1.0806x; 1.0806x over previous
import math

import jax
import jax.numpy as jnp
from jax.experimental import pallas as pl

B = 64
T = 1024
D = 256
BT = 8
NSPLIT = 4
TS = T // NSPLIT


def _ln(x, g, b, eps=1e-5):
    mu = jnp.mean(x, axis=-1, keepdims=True)
    var = jnp.mean((x - mu) * (x - mu), axis=-1, keepdims=True)
    return (x - mu) * jax.lax.rsqrt(var + eps) * g + b


def _body(*refs):
    kv_refs = refs[:2 * NSPLIT]
    (q_ref, Wq_ref, Wk_ref, Wv_ref, Wo_ref,
     bq_ref, bv_ref, bo_ref, gq_ref, bqn_ref, go_ref, bon_ref,
     out_ref) = refs[2 * NSPLIT:]
    k_refs = kv_refs[:NSPLIT]
    v_refs = kv_refs[NSPLIT:]

    qn = _ln(q_ref[:], gq_ref[:], bqn_ref[:])
    qp = jax.lax.dot_general(qn, Wq_ref[:], (((1,), (1,)), ((), ())),
                             preferred_element_type=jnp.float32) + bq_ref[:]
    qk = jax.lax.dot_general(qp, Wk_ref[:], (((1,), (0,)), ((), ())),
                             preferred_element_type=jnp.float32)
    qk = qk * (1.0 / math.sqrt(D))

    dn = (((1,), (2,)), ((0,), (0,)))
    ss = [jax.lax.dot_general(qk, kr[:], dn,
                              preferred_element_type=jnp.float32)
          for kr in k_refs]
    m = ss[0].max(axis=1, keepdims=True)
    for si in ss[1:]:
        m = jnp.maximum(m, si.max(axis=1, keepdims=True))
    es = [jnp.exp(si - m) for si in ss]
    s = es[0].sum(axis=1, keepdims=True)
    for ei in es[1:]:
        s = s + ei.sum(axis=1, keepdims=True)

    dn2 = (((1,), (1,)), ((0,), (0,)))
    rv = jax.lax.dot_general(es[0], v_refs[0][:], dn2,
                             preferred_element_type=jnp.float32)
    for ei, vr in zip(es[1:], v_refs[1:]):
        rv = rv + jax.lax.dot_general(ei, vr[:], dn2,
                                      preferred_element_type=jnp.float32)
    rv = rv * (1.0 / s)

    ret = jax.lax.dot_general(rv, Wv_ref[:], (((1,), (1,)), ((), ())),
                              preferred_element_type=jnp.float32) + bv_ref[:]
    out = jax.lax.dot_general(ret, Wo_ref[:], (((1,), (1,)), ((), ())),
                              preferred_element_type=jnp.float32) + bo_ref[:]
    out_ref[:] = _ln(out, go_ref[:], bon_ref[:])


def kernel(keys_in, values_in, query, Wq, bq, Wk, bk, Wv, bv, Wo, bo,
           g_q, b_qn, g_o, b_on):
    del bk
    vecs = [v.reshape(1, D) for v in (bq, bv, bo, g_q, b_qn, g_o, b_on)]
    full = pl.BlockSpec((1, D), lambda i: (0, 0))
    mat = pl.BlockSpec((D, D), lambda i: (0, 0))

    def win(j):
        return pl.BlockSpec((BT, TS, D), lambda i, j=j: (i, j, 0))

    kv_specs = [win(j) for j in range(NSPLIT)] * 2
    out = pl.pallas_call(
        _body,
        grid=(B // BT,),
        in_specs=kv_specs + [
            pl.BlockSpec((BT, D), lambda i: (i, 0)),
            mat, mat, mat, mat,
            full, full, full, full, full, full, full,
        ],
        out_specs=pl.BlockSpec((BT, D), lambda i: (i, 0)),
        out_shape=jax.ShapeDtypeStruct((B, D), jnp.float32),
    )(*([keys_in] * NSPLIT), *([values_in] * NSPLIT), query,
      Wq, Wk, Wv, Wo, *vecs)
    return out
